# Initial kernel scaffold; baseline (speedup 1.0000x reference)
import jax
import jax.numpy as jnp
from jax.experimental import pallas as pl


def kernel(x_q, edge_index_q, x_c, edge_index_c, graph_sizes, W1, b1, W2, b2, W3, b3, W_att, A_ntn, W_b, ntn_bias, W_fc1, b_fc1, W_fc2, b_fc2):
    raise NotImplementedError("write your pallas kernel here")


# speedup vs baseline: 27.9167x; 27.9167x over previous
import functools

import jax
import jax.numpy as jnp
from jax import lax
from jax.experimental import pallas as pl
from jax.experimental.pallas import tpu as pltpu
from jax.experimental.pallas import tpu_sc as plsc

N = 10000
E = 320000
NC = 2
NS = 16
NW = NC * NS
C = 80
NCH = E // (NW * C)
assert NCH % 2 == 1 and NCH * NW * C == E
NCH2 = 2 * E // (NW * C)
ROWS_PER_TILE = 632
NPAD = ROWS_PER_TILE * NS
DROWS_PER_TILE = 1256
DPAD = DROWS_PER_TILE * NS

_mesh = lambda: plsc.VectorSubcoreMesh(core_axis_name="c", subcore_axis_name="s")


def _deg_body(dst_hbm, ones_hbm, zeros_hbm, out_hbm, dstv, onesv, acc, sem):
    c = lax.axis_index("c")
    s = lax.axis_index("s")
    pltpu.sync_copy(dst_hbm.at[c, s], dstv)
    pltpu.sync_copy(ones_hbm, onesv)
    pltpu.sync_copy(zeros_hbm, acc.at[pl.ds(s * DROWS_PER_TILE, DROWS_PER_TILE)])
    plsc.subcore_barrier()

    def body(j, carry):
        pltpu.sync_copy(onesv, acc.at[dstv.at[j]], add=True)
        return carry

    lax.fori_loop(0, NCH2, body, 0)
    plsc.subcore_barrier()
    rows = pl.ds(s * DROWS_PER_TILE, DROWS_PER_TILE)
    pltpu.sync_copy(acc.at[rows], out_hbm.at[c, rows])


def _degree_partials(dst_all):
    ones = jnp.ones((C, 16), jnp.float32)
    zeros = jnp.zeros((DROWS_PER_TILE, 16), jnp.float32)
    f = pl.kernel(
        _deg_body,
        out_type=jax.ShapeDtypeStruct((NC, DPAD, 16), jnp.float32),
        mesh=_mesh(),
        compiler_params=pltpu.CompilerParams(use_tc_tiling_on_sc=False),
        scratch_types=[
            pltpu.VMEM((NCH2, C), jnp.int32),
            pltpu.VMEM((C, 16), jnp.float32),
            pltpu.VMEM_SHARED((DPAD, 16), jnp.float32),
            pltpu.SemaphoreType.DMA,
        ],
    )
    return f(dst_all, ones, zeros)


def _seg_body(h_hbm, src_hbm, dst_hbm, zeros_hbm, out_hbm,
              srcv, dstv, rows0, rows1, acc, sem0, sem1):
    c = lax.axis_index("c")
    s = lax.axis_index("s")
    pltpu.sync_copy(src_hbm.at[c, s], srcv)
    pltpu.sync_copy(dst_hbm.at[c, s], dstv)
    pltpu.sync_copy(zeros_hbm, acc.at[pl.ds(s * ROWS_PER_TILE, ROWS_PER_TILE)])
    plsc.subcore_barrier()

    pltpu.async_copy(h_hbm.at[srcv.at[0]], rows0, sem0)

    def wait(buf, sem):
        pltpu.make_async_copy(h_hbm.at[srcv.at[0]], buf, sem).wait()

    def body(i, carry):
        j = 2 * i
        pltpu.async_copy(h_hbm.at[srcv.at[j + 1]], rows1, sem1)
        wait(rows0, sem0)
        pltpu.sync_copy(rows0, acc.at[dstv.at[j]], add=True)
        pltpu.async_copy(h_hbm.at[srcv.at[j + 2]], rows0, sem0)
        wait(rows1, sem1)
        pltpu.sync_copy(rows1, acc.at[dstv.at[j + 1]], add=True)
        return carry

    lax.fori_loop(0, NCH // 2, body, 0)
    wait(rows0, sem0)
    pltpu.sync_copy(rows0, acc.at[dstv.at[NCH - 1]], add=True)

    plsc.subcore_barrier()
    rows = pl.ds(s * ROWS_PER_TILE, ROWS_PER_TILE)
    pltpu.sync_copy(acc.at[rows], out_hbm.at[c, rows])


def _segment_partials(h, src4, dst4, F):
    zeros = jnp.zeros((ROWS_PER_TILE, F), jnp.float32)
    f = pl.kernel(
        _seg_body,
        out_type=jax.ShapeDtypeStruct((NC, NPAD, F), jnp.float32),
        mesh=_mesh(),
        compiler_params=pltpu.CompilerParams(use_tc_tiling_on_sc=False),
        scratch_types=[
            pltpu.VMEM((NCH, C), jnp.int32),
            pltpu.VMEM((NCH, C), jnp.int32),
            pltpu.VMEM((C, F), jnp.float32),
            pltpu.VMEM((C, F), jnp.float32),
            pltpu.VMEM_SHARED((NPAD, F), jnp.float32),
            pltpu.SemaphoreType.DMA,
            pltpu.SemaphoreType.DMA,
        ],
    )
    return f(h, src4, dst4, zeros)


def _bf(a):
    return a.astype(jnp.bfloat16).astype(jnp.float32)


def _dot(a, b):
    return jnp.dot(_bf(a), _bf(b), preferred_element_type=jnp.float32,
                   precision=lax.Precision.HIGHEST)


def _dinv_body(p_ref, o_ref):
    deg = 1.0 + p_ref[0, :2 * N, 0:1] + p_ref[1, :2 * N, 0:1]
    o_ref[...] = lax.rsqrt(deg)


def _dinv(p):
    return pl.pallas_call(
        _dinv_body,
        out_shape=jax.ShapeDtypeStruct((2 * N, 1), jnp.float32),
    )(p)


def _mm_body(x_ref, w_ref, d_ref, o_ref):
    o_ref[...] = _dot(x_ref[...], w_ref[...]) * d_ref[...]


def _matmul_scale(x, W, dinv):
    return pl.pallas_call(
        _mm_body,
        out_shape=jax.ShapeDtypeStruct((x.shape[0], W.shape[1]), jnp.float32),
    )(x, W, dinv)


def _combine_body(do_relu, p_ref, h_ref, d_ref, b_ref, o_ref):
    y = d_ref[...] * (p_ref[0, :N] + p_ref[1, :N] + h_ref[...]) + b_ref[...]
    if do_relu:
        y = jnp.maximum(y, 0.0)
    o_ref[...] = _bf(y)


def _combine(p, h, dinv, b, do_relu):
    return pl.pallas_call(
        functools.partial(_combine_body, do_relu),
        out_shape=jax.ShapeDtypeStruct(h.shape, jnp.float32),
    )(p, h, dinv, b)


def _tail_body(eq_ref, ec_ref, gs_ref, watt_ref, a_ref, wb_ref, bias_ref,
               wfc1_ref, bfc1_ref, wfc2_ref, bfc2_ref, o_ref):
    gs = gs_ref[...]

    def pool(emb):
        m = jnp.sum(emb, axis=1)
        ctx = _bf(jnp.tanh(_dot(m, watt_ref[...]) / gs))
        s = _bf(jax.nn.sigmoid(jnp.sum(emb * ctx[:, None, :], axis=2)))
        return jnp.sum(emb * s[:, :, None], axis=1)

    e1 = pool(eq_ref[...])
    e2 = pool(ec_ref[...])
    outer = _bf(e1[:, :, None] * e2[:, None, :])
    cols = []
    for k in range(16):
        t = jnp.sum(outer * _bf(a_ref[k]), axis=2)
        cols.append(jnp.sum(t, axis=1, keepdims=True))
    ntn = jnp.concatenate(cols, axis=1)
    cat = jnp.concatenate([e1, e2], axis=1)
    scores = ntn + _dot(cat, wb_ref[...]) + bias_ref[...]
    scores = jnp.maximum(scores, 0.0)
    h = jnp.maximum(_dot(scores, wfc1_ref[...]) + bfc1_ref[...], 0.0)
    p = jax.nn.sigmoid(_dot(h, wfc2_ref[...]) + bfc2_ref[...])
    o_ref[...] = -gs * jnp.log(p)


def _tail(eq, ec, gs, W_att, A_ntn, W_b, ntn_bias, W_fc1, b_fc1, W_fc2, b_fc2):
    B = gs.shape[0]
    return pl.pallas_call(
        _tail_body,
        out_shape=jax.ShapeDtypeStruct((B, 1), jnp.float32),
    )(eq, ec, gs, W_att, A_ntn, W_b, ntn_bias.reshape(1, -1),
      W_fc1, b_fc1.reshape(1, -1), W_fc2, b_fc2.reshape(1, -1))


def kernel(x_q, edge_index_q, x_c, edge_index_c, graph_sizes, W1, b1, W2, b2,
           W3, b3, W_att, A_ntn, W_b, ntn_bias, W_fc1, b_fc1, W_fc2, b_fc2):
    src_q = edge_index_q[0].reshape(NC, NS, NCH, C)
    dst_q = edge_index_q[1].reshape(NC, NS, NCH, C)
    src_c = edge_index_c[0].reshape(NC, NS, NCH, C)
    dst_c = edge_index_c[1].reshape(NC, NS, NCH, C)

    dst_all = jnp.concatenate(
        [edge_index_q[1], edge_index_c[1] + N]).reshape(NC, NS, NCH2, C)
    dp = _degree_partials(dst_all)
    dinv_all = _dinv(dp)
    dinv_q = dinv_all[:N]
    dinv_c = dinv_all[N:]

    def gnn(x, src4, dst4, dinv):
        h = _matmul_scale(x, W1, dinv)
        p = _segment_partials(h, src4, dst4, 64)
        y = _combine(p, h, dinv, b1.reshape(1, -1), True)
        h = _matmul_scale(y, W2, dinv)
        p = _segment_partials(h, src4, dst4, 32)
        y = _combine(p, h, dinv, b2.reshape(1, -1), True)
        h = _matmul_scale(y, W3, dinv)
        p = _segment_partials(h, src4, dst4, 16)
        return _combine(p, h, dinv, b3.reshape(1, -1), False)

    yq = gnn(x_q, src_q, dst_q, dinv_q)
    yc = gnn(x_c, src_c, dst_c, dinv_c)

    B = graph_sizes.shape[0]
    eq = yq.reshape(B, N // B, 16)
    ec = yc.reshape(B, N // B, 16)
    ged = _tail(eq, ec, graph_sizes.reshape(B, 1), W_att, A_ntn, W_b,
                ntn_bias, W_fc1, b_fc1, W_fc2, b_fc2)
    return ged[:, 0]


# speedup vs baseline: 37.2499x; 1.3343x over previous
import functools

import jax
import jax.numpy as jnp
from jax import lax
from jax.experimental import pallas as pl
from jax.experimental.pallas import tpu as pltpu
from jax.experimental.pallas import tpu_sc as plsc

N = 10000
E = 320000
NC = 2
NS = 16
NW = NC * NS
C = 125
NSLOTS = E // (NS * C)
assert NSLOTS % 8 == 0 and NSLOTS * NS * C == E
NPAD = 10112
RPT = NPAD // NS

_mesh = lambda: plsc.VectorSubcoreMesh(core_axis_name="c", subcore_axis_name="s")
_sc_params = lambda: pltpu.CompilerParams(use_tc_tiling_on_sc=False)


def _deg_body(dst_hbm, ones_hbm, zeros_hbm, out_hbm, dstv, onesv, acc, sem):
    c = lax.axis_index("c")
    s = lax.axis_index("s")
    pltpu.sync_copy(dst_hbm.at[c, s], dstv)
    pltpu.sync_copy(ones_hbm, onesv)
    pltpu.sync_copy(zeros_hbm, acc.at[pl.ds(s * RPT, RPT)])
    plsc.subcore_barrier()

    def body(i, carry):
        for b in range(8):
            pltpu.async_copy(onesv, acc.at[dstv.at[8 * i + b]], sem, add=True)
        for b in range(8):
            pltpu.make_async_copy(onesv, acc.at[dstv.at[0]], sem).wait()
        return carry

    lax.fori_loop(0, NSLOTS // 8, body, 0)
    plsc.subcore_barrier()
    rows = pl.ds(s * RPT, RPT)
    pltpu.sync_copy(acc.at[rows], out_hbm.at[c, rows])


def _degree_partials(dst_all):
    ones = jnp.ones((C, 16), jnp.float32)
    zeros = jnp.zeros((RPT, 16), jnp.float32)
    f = pl.kernel(
        _deg_body,
        out_type=jax.ShapeDtypeStruct((NC, NPAD, 16), jnp.float32),
        mesh=_mesh(),
        compiler_params=_sc_params(),
        scratch_types=[
            pltpu.VMEM((NSLOTS, C), jnp.int32),
            pltpu.VMEM((C, 16), jnp.float32),
            pltpu.VMEM_SHARED((NPAD, 16), jnp.float32),
            pltpu.SemaphoreType.DMA,
        ],
    )
    return f(dst_all, ones, zeros)


def _seg_body(h_hbm, src_hbm, dst_hbm, zeros_hbm, out_hbm,
              srcv, dstv, r0, r1, r2, r3, acc,
              g0, g1, g2, g3, s0, s1, s2, s3):
    c = lax.axis_index("c")
    s = lax.axis_index("s")
    pltpu.sync_copy(src_hbm.at[c, s], srcv)
    pltpu.sync_copy(dst_hbm.at[c, s], dstv)
    pltpu.sync_copy(zeros_hbm, acc.at[pl.ds(s * RPT, RPT)])
    plsc.subcore_barrier()

    rows = [r0, r1, r2, r3]
    gsem = [g0, g1, g2, g3]
    ssem = [s0, s1, s2, s3]

    def g_start(j, b):
        pltpu.async_copy(h_hbm.at[srcv.at[j]], rows[b], gsem[b])

    def g_wait(b):
        pltpu.make_async_copy(h_hbm.at[srcv.at[0]], rows[b], gsem[b]).wait()

    def s_start(j, b):
        pltpu.async_copy(rows[b], acc.at[dstv.at[j]], ssem[b], add=True)

    def s_wait(b):
        pltpu.make_async_copy(rows[b], acc.at[dstv.at[0]], ssem[b]).wait()

    g_start(0, 0)
    g_start(1, 1)
    g_start(2, 2)
    g_wait(0); s_start(0, 0); g_start(3, 3)

    def body(i, carry):
        j = 4 * i + 1
        for off in range(4):
            b = (1 + off) % 4
            p = (b - 1) % 4
            g_wait(b)
            s_start(j + off, b)
            s_wait(p)
            g_start(j + off + 3, p)
        return carry

    lax.fori_loop(0, (NSLOTS - 4) // 4, body, 0)
    for jj in range(NSLOTS - 3, NSLOTS):
        b = jj % 4
        g_wait(b)
        s_start(jj, b)
        s_wait((b - 1) % 4)
    s_wait((NSLOTS - 1) % 4)

    plsc.subcore_barrier()
    rows_sl = pl.ds(s * RPT, RPT)
    pltpu.sync_copy(acc.at[rows_sl], out_hbm.at[c, rows_sl])


def _segment_partials(h, src4, dst4, F):
    zeros = jnp.zeros((RPT, F), jnp.float32)
    f = pl.kernel(
        _seg_body,
        out_type=jax.ShapeDtypeStruct((NC, NPAD, F), jnp.float32),
        mesh=_mesh(),
        compiler_params=_sc_params(),
        scratch_types=[
            pltpu.VMEM((NSLOTS, C), jnp.int32),
            pltpu.VMEM((NSLOTS, C), jnp.int32),
            pltpu.VMEM((C, F), jnp.float32),
            pltpu.VMEM((C, F), jnp.float32),
            pltpu.VMEM((C, F), jnp.float32),
            pltpu.VMEM((C, F), jnp.float32),
            pltpu.VMEM_SHARED((NPAD, F), jnp.float32),
        ] + [pltpu.SemaphoreType.DMA] * 8,
    )
    return f(h, src4, dst4, zeros)


def _bf(a):
    u = lax.bitcast_convert_type(a, jnp.uint32)
    r = u + jnp.uint32(0x7FFF) + ((u >> 16) & jnp.uint32(1))
    return lax.bitcast_convert_type(r & jnp.uint32(0xFFFF0000), jnp.float32)


def _dot(a, b):
    return jnp.dot(a.astype(jnp.bfloat16), b.astype(jnp.bfloat16),
                   preferred_element_type=jnp.float32)


def _dinv_body(p_ref, o_ref):
    dinv = lax.rsqrt(1.0 + jnp.concatenate(
        [p_ref[0, :N, 0:1], p_ref[1, :N, 0:1]], axis=0))
    o_ref[...] = dinv


def _dinv(p):
    return pl.pallas_call(
        _dinv_body,
        out_shape=jax.ShapeDtypeStruct((2 * N, 1), jnp.float32),
    )(p)


def _mm_body(x_ref, w_ref, d_ref, o_ref):
    o_ref[...] = _dot(x_ref[...], w_ref[...]) * d_ref[...]


_BM = 2000


def _matmul_scale(x, W, dinv):
    M, Fin = x.shape
    Fout = W.shape[1]
    return pl.pallas_call(
        _mm_body,
        grid=(M // _BM,),
        in_specs=[
            pl.BlockSpec((_BM, Fin), lambda m: (m, 0)),
            pl.BlockSpec((Fin, Fout), lambda m: (0, 0)),
            pl.BlockSpec((_BM, 1), lambda m: (m, 0)),
        ],
        out_specs=pl.BlockSpec((_BM, Fout), lambda m: (m, 0)),
        out_shape=jax.ShapeDtypeStruct((M, Fout), jnp.float32),
    )(x, W, dinv)


def _combine_body(do_relu, p_ref, h_ref, d_ref, b_ref, o_ref):
    y = d_ref[...] * (p_ref[0] + h_ref[...]) + b_ref[...]
    if do_relu:
        y = jnp.maximum(y, 0.0)
    o_ref[...] = _bf(y)


_NBG = N // _BM


def _combine(p, h, dinv, b, do_relu):
    M, F = h.shape
    return pl.pallas_call(
        functools.partial(_combine_body, do_relu),
        grid=(M // _BM,),
        in_specs=[
            pl.BlockSpec((1, _BM, F), lambda m: (m // _NBG, m % _NBG, 0)),
            pl.BlockSpec((_BM, F), lambda m: (m, 0)),
            pl.BlockSpec((_BM, 1), lambda m: (m, 0)),
            pl.BlockSpec((1, F), lambda m: (0, 0)),
        ],
        out_specs=pl.BlockSpec((_BM, F), lambda m: (m, 0)),
        out_shape=jax.ShapeDtypeStruct((M, F), jnp.float32),
    )(p, h, dinv, b)


def _tail_body(eq_ref, ec_ref, gs_ref, watt_ref, a_ref, wb_ref, bias_ref,
               wfc1_ref, bfc1_ref, wfc2_ref, bfc2_ref, o_ref):
    gs = gs_ref[...]

    def pool(emb):
        P = _dot(emb.reshape(-1, 16), watt_ref[...]).reshape(emb.shape)
        ctx = _bf(jnp.tanh(jnp.sum(P, axis=1) / gs))
        s = _bf(jax.nn.sigmoid(jnp.sum(emb * ctx[:, None, :], axis=2)))
        return jnp.sum(emb * s[:, :, None], axis=1)

    e1 = pool(eq_ref[...])
    e2 = pool(ec_ref[...])
    outer = _bf(e1[:, :, None] * e2[:, None, :])
    cols = []
    for k in range(16):
        t = jnp.sum(outer * _bf(a_ref[k]), axis=2)
        cols.append(jnp.sum(t, axis=1, keepdims=True))
    ntn = jnp.concatenate(cols, axis=1)
    cat = jnp.concatenate([e1, e2], axis=1)
    scores = ntn + _dot(cat, wb_ref[...]) + bias_ref[...]
    scores = jnp.maximum(scores, 0.0)
    h = jnp.maximum(_dot(scores, wfc1_ref[...]) + bfc1_ref[...], 0.0)
    p = jax.nn.sigmoid(_dot(h, wfc2_ref[...]) + bfc2_ref[...])
    o_ref[...] = -gs * jnp.log(p)


def _tail(eq, ec, gs, W_att, A_ntn, W_b, ntn_bias, W_fc1, b_fc1, W_fc2, b_fc2):
    B = gs.shape[0]
    return pl.pallas_call(
        _tail_body,
        out_shape=jax.ShapeDtypeStruct((B, 1), jnp.float32),
    )(eq, ec, gs, W_att, A_ntn, W_b, ntn_bias.reshape(1, -1),
      W_fc1, b_fc1.reshape(1, -1), W_fc2, b_fc2.reshape(1, -1))


def kernel(x_q, edge_index_q, x_c, edge_index_c, graph_sizes, W1, b1, W2, b2,
           W3, b3, W_att, A_ntn, W_b, ntn_bias, W_fc1, b_fc1, W_fc2, b_fc2):
    src4 = jnp.stack([edge_index_q[0].reshape(NS, NSLOTS, C),
                      (edge_index_c[0] + N).reshape(NS, NSLOTS, C)])
    dst4 = jnp.stack([edge_index_q[1].reshape(NS, NSLOTS, C),
                      edge_index_c[1].reshape(NS, NSLOTS, C)])

    dp = _degree_partials(dst4)
    dinv = _dinv(dp)

    x = jnp.concatenate([x_q, x_c], axis=0)

    def layer(x, W, b, F, do_relu):
        h = _matmul_scale(x, W, dinv)
        p = _segment_partials(h, src4, dst4, F)
        return _combine(p, h, dinv, b.reshape(1, -1), do_relu)

    y = layer(x, W1, b1, 64, True)
    y = layer(y, W2, b2, 32, True)
    y = layer(y, W3, b3, 16, False)

    B = graph_sizes.shape[0]
    eq = y[:N].reshape(B, N // B, 16)
    ec = y[N:].reshape(B, N // B, 16)
    ged = _tail(eq, ec, graph_sizes.reshape(B, 1), W_att, A_ntn, W_b,
                ntn_bias, W_fc1, b_fc1, W_fc2, b_fc2)
    return ged[:, 0]
